# baseline (speedup 1.0000x reference)
import functools

import jax
import jax.numpy as jnp
from jax import lax
from jax.experimental import pallas as pl
from jax.experimental.pallas import tpu as pltpu
from jax.experimental.pallas import tpu_sc as plsc

B = 16384
D = 16
V = 64
L = 16


def kernel(x, q_table):
    info = plsc.get_sparse_core_info()
    nc, ns = info.num_cores, info.num_subcores
    nw = nc * ns
    bpw = B // nw

    mesh = plsc.VectorSubcoreMesh(core_axis_name="c", subcore_axis_name="s")

    dnums = lax.GatherDimensionNumbers(
        offset_dims=(), collapsed_slice_dims=(0,), start_index_map=(0,))

    def lane_gather(v, idx):
        return lax.gather(
            v, idx[:, None], dimension_numbers=dnums, slice_sizes=(1,),
            mode=lax.GatherScatterMode.PROMISE_IN_BOUNDS)

    @functools.partial(
        pl.kernel,
        mesh=mesh,
        out_type=jax.ShapeDtypeStruct((D, B), jnp.float32),
        scratch_types=[
            pltpu.VMEM((bpw,), jnp.int32),
            pltpu.VMEM((D, V), jnp.float32),
            pltpu.VMEM((D, bpw), jnp.float32),
            pltpu.SemaphoreType.DMA,
            pltpu.SemaphoreType.DMA,
            pltpu.SemaphoreType.DMA,
        ],
    )
    def k(idx_hbm, tab_hbm, out_hbm, idx_v, tab_v, cols_v,
          sem_i, sem_t, sem_o):
        wid = lax.axis_index("s") * nc + lax.axis_index("c")
        base = wid * bpw
        with jax.named_scope("in_dma"):
            cp_t = pltpu.async_copy(tab_hbm, tab_v, sem_t)
            cp_i = pltpu.async_copy(idx_hbm.at[pl.ds(base, bpw)], idx_v, sem_i)
            cp_i.wait()
            cp_t.wait()

        CB = 4
        with jax.named_scope("lookup_loop"):
            for p in range(D // CB):
                cs = range(p * CB, (p + 1) * CB)
                t = {c: [tab_v[c, pl.ds(h * L, L)] for h in range(V // L)]
                     for c in cs}

                def group(g, _, cs=cs, t=t):
                    v = idx_v[pl.ds(g * L, L)]
                    lo = v & (L - 1)
                    hi = v >> 4
                    for c in cs:
                        acc = lane_gather(t[c][0], lo)
                        for h in range(1, V // L):
                            acc = jnp.where(hi == h, lane_gather(t[c][h], lo),
                                            acc)
                        cols_v[c, pl.ds(g * L, L)] = acc
                    return 0

                lax.fori_loop(0, bpw // L, group, 0)
                pltpu.async_copy(
                    cols_v.at[pl.ds(p * CB, CB)],
                    out_hbm.at[pl.ds(p * CB, CB), pl.ds(base, bpw)], sem_o)
        with jax.named_scope("out_dma"):
            for p in range(D // CB):
                pltpu.make_async_copy(
                    cols_v.at[pl.ds(p * CB, CB)],
                    out_hbm.at[pl.ds(p * CB, CB), pl.ds(base, bpw)],
                    sem_o).wait()

    out_t = k(x[:, 0], q_table.T)
    return out_t.T


# speedup vs baseline: 1.0232x; 1.0232x over previous
import functools

import jax
import jax.numpy as jnp
from jax import lax
from jax.experimental import pallas as pl
from jax.experimental.pallas import tpu as pltpu
from jax.experimental.pallas import tpu_sc as plsc

B = 16384
D = 16
V = 64
L = 16


def kernel(x, q_table):
    info = plsc.get_sparse_core_info()
    nc, ns = info.num_cores, info.num_subcores
    nw = nc * ns
    bpw = B // nw

    mesh = plsc.VectorSubcoreMesh(core_axis_name="c", subcore_axis_name="s")

    dnums = lax.GatherDimensionNumbers(
        offset_dims=(), collapsed_slice_dims=(0,), start_index_map=(0,))

    def lane_gather(v, idx):
        return lax.gather(
            v, idx[:, None], dimension_numbers=dnums, slice_sizes=(1,),
            mode=lax.GatherScatterMode.PROMISE_IN_BOUNDS)

    @functools.partial(
        pl.kernel,
        mesh=mesh,
        out_type=jax.ShapeDtypeStruct((D, B), jnp.float32),
        scratch_types=[
            pltpu.VMEM((bpw,), jnp.int32),
            pltpu.VMEM((D, V), jnp.float32),
            pltpu.VMEM((D, bpw), jnp.float32),
            pltpu.SemaphoreType.DMA,
            [pltpu.SemaphoreType.DMA] * 4,
            pltpu.SemaphoreType.DMA,
        ],
    )
    def k(idx_hbm, tab_hbm, out_hbm, idx_v, tab_v, cols_v,
          sem_i, sem_t, sem_o):
        wid = lax.axis_index("s") * nc + lax.axis_index("c")
        base = wid * bpw
        with jax.named_scope("in_dma"):
            cp_t = [
                pltpu.async_copy(tab_hbm.at[pl.ds(p * 4, 4)],
                                 tab_v.at[pl.ds(p * 4, 4)], sem_t[p])
                for p in range(4)
            ]
            cp_i = pltpu.async_copy(idx_hbm.at[pl.ds(base, bpw)], idx_v, sem_i)
            cp_i.wait()

        CB = 4
        with jax.named_scope("lookup_loop"):
            for p in range(D // CB):
                cp_t[p].wait()
                cs = range(p * CB, (p + 1) * CB)
                t = {c: [tab_v[c, pl.ds(h * L, L)] for h in range(V // L)]
                     for c in cs}

                def group(g, _, cs=cs, t=t):
                    v = idx_v[pl.ds(g * L, L)]
                    lo = v & (L - 1)
                    hi = v >> 4
                    for c in cs:
                        acc = lane_gather(t[c][0], lo)
                        for h in range(1, V // L):
                            acc = jnp.where(hi == h, lane_gather(t[c][h], lo),
                                            acc)
                        cols_v[c, pl.ds(g * L, L)] = acc
                    return 0

                lax.fori_loop(0, bpw // L, group, 0)
                pltpu.async_copy(
                    cols_v.at[pl.ds(p * CB, CB)],
                    out_hbm.at[pl.ds(p * CB, CB), pl.ds(base, bpw)], sem_o)
        with jax.named_scope("out_dma"):
            for p in range(D // CB):
                pltpu.make_async_copy(
                    cols_v.at[pl.ds(p * CB, CB)],
                    out_hbm.at[pl.ds(p * CB, CB), pl.ds(base, bpw)],
                    sem_o).wait()

    out_t = k(x[:, 0], q_table.T)
    return out_t.T
